# baseline (speedup 1.0000x reference)
import jax
import jax.numpy as jnp
from jax import lax
from jax.experimental import pallas as pl

B, S, K, D = 4096, 20, 64, 32
BB = 128


def _body(s_ref, m_ref, k_ref, u_ref, v_ref, w_ref, o_ref):
    k = k_ref[...]
    u = u_ref[...]
    v = v_ref[...]
    w = w_ref[...]

    f32 = jnp.float32
    kv = jnp.dot(k.reshape(BB * K, D), v, preferred_element_type=f32)
    kv = kv.reshape(BB, K, D)

    def step(t, h):
        s_t = s_ref[pl.ds(t, 1)].reshape(BB, D)
        m_t = m_ref[pl.ds(t, 1)].reshape(BB, 1)
        g_logit = jnp.sum(s_t[:, None, :] * (h + k), axis=2)
        gates = jax.nn.sigmoid(g_logit)
        hu = jnp.dot(h.reshape(BB * K, D), u, preferred_element_type=f32)
        sw = jnp.dot(s_t, w, preferred_element_type=f32)
        h_tilda = jax.nn.sigmoid(hu.reshape(BB, K, D) + kv + sw[:, None, :])
        upd = h + gates[:, :, None] * h_tilda
        sq = jnp.sum(upd * upd, axis=2, keepdims=True)
        upd = upd * lax.rsqrt(jnp.maximum(sq, 1e-12))
        return jnp.where(m_t[:, :, None] > 0.5, upd, h)

    o_ref[...] = lax.fori_loop(0, S, step, k)


@jax.jit
def kernel(encoded_sents, mask, keys, U, V, W):
    s_tm = jnp.swapaxes(encoded_sents, 0, 1)
    m_tm = jnp.swapaxes(mask, 0, 1).astype(jnp.float32)[:, :, None]
    grid = (B // BB,)
    return pl.pallas_call(
        _body,
        grid=grid,
        in_specs=[
            pl.BlockSpec((S, BB, D), lambda i: (0, i, 0)),
            pl.BlockSpec((S, BB, 1), lambda i: (0, i, 0)),
            pl.BlockSpec((BB, K, D), lambda i: (i, 0, 0)),
            pl.BlockSpec((D, D), lambda i: (0, 0)),
            pl.BlockSpec((D, D), lambda i: (0, 0)),
            pl.BlockSpec((D, D), lambda i: (0, 0)),
        ],
        out_specs=pl.BlockSpec((BB, K, D), lambda i: (i, 0, 0)),
        out_shape=jax.ShapeDtypeStruct((B, K, D), jnp.float32),
    )(s_tm, m_tm, keys, U, V, W)


# speedup vs baseline: 2.9551x; 2.9551x over previous
import jax
import jax.numpy as jnp
from jax import lax
from jax.experimental import pallas as pl

B, S, K, D = 4096, 20, 64, 32
BB = 128


def _body(s_ref, m_ref, k_ref, ut_ref, vt_ref, wt_ref, o_ref):
    kt = k_ref[...].reshape(D, K * BB)
    ut = ut_ref[...]
    vt = vt_ref[...]
    wt = wt_ref[...]

    f32 = jnp.float32
    kv = jnp.dot(vt, kt, preferred_element_type=f32)
    kt3 = kt.reshape(D, K, BB)
    kv3 = kv.reshape(D, K, BB)

    def step(t, h):
        h3 = h.reshape(D, K, BB)
        s_t = s_ref[pl.ds(t, 1)].reshape(D, BB)
        m_t = m_ref[pl.ds(t, 1)].reshape(1, 1, BB)
        g_logit = jnp.sum(s_t[:, None, :] * (h3 + kt3), axis=0)
        gates = jax.nn.sigmoid(g_logit)
        hu = jnp.dot(ut, h, preferred_element_type=f32)
        sw = jnp.dot(wt, s_t, preferred_element_type=f32)
        h_tilda = jax.nn.sigmoid(hu.reshape(D, K, BB) + kv3 + sw[:, None, :])
        upd = h3 + gates[None, :, :] * h_tilda
        sq = jnp.sum(upd * upd, axis=0)
        upd = upd * lax.rsqrt(jnp.maximum(sq, 1e-12))[None, :, :]
        return jnp.where(m_t > 0.5, upd, h3).reshape(D, K * BB)

    o_ref[...] = lax.fori_loop(0, S, step, kt).reshape(D, K, BB)


@jax.jit
def kernel(encoded_sents, mask, keys, U, V, W):
    s_t = jnp.transpose(encoded_sents, (1, 2, 0))
    m_t = jnp.swapaxes(mask, 0, 1).astype(jnp.float32)[:, None, :]
    k_t = jnp.transpose(keys, (2, 1, 0))
    grid = (B // BB,)
    out_t = pl.pallas_call(
        _body,
        grid=grid,
        in_specs=[
            pl.BlockSpec((S, D, BB), lambda i: (0, 0, i)),
            pl.BlockSpec((S, 1, BB), lambda i: (0, 0, i)),
            pl.BlockSpec((D, K, BB), lambda i: (0, 0, i)),
            pl.BlockSpec((D, D), lambda i: (0, 0)),
            pl.BlockSpec((D, D), lambda i: (0, 0)),
            pl.BlockSpec((D, D), lambda i: (0, 0)),
        ],
        out_specs=pl.BlockSpec((D, K, BB), lambda i: (0, 0, i)),
        out_shape=jax.ShapeDtypeStruct((D, K, B), jnp.float32),
    )(s_t, m_t, k_t, U.T, V.T, W.T)
    return jnp.transpose(out_t, (2, 1, 0))
